# baseline (speedup 1.0000x reference)
import functools

import jax
import jax.numpy as jnp
from jax import lax
from jax.experimental import pallas as pl
from jax.experimental.pallas import tpu as pltpu
from jax.experimental.pallas import tpu_sc as plsc

_N = 10000
_E = 320000
_D = 128
_H = 32
_G = 64
_C = 2

_NTILES = 32
_CH = 128
_TCHUNKS = 2560
_EPAD = _TCHUNKS * _CH
_KW = 160
_WCORE = 1
_NB = 8
_NPAD = 10112
_RPT = _NPAD // 16


def _segment_sum_sc(p, src3, dst3, zeros):
  mesh = plsc.VectorSubcoreMesh(core_axis_name="c", subcore_axis_name="s")

  @functools.partial(
      pl.kernel,
      mesh=mesh,
      compiler_params=pltpu.CompilerParams(use_tc_tiling_on_sc=False),
      out_type=jax.ShapeDtypeStruct((_NPAD, _H), jnp.float32),
      scratch_types=(
          [pltpu.VMEM((_KW, _CH), jnp.int32),
           pltpu.VMEM((_KW, _CH), jnp.int32)]
          + [pltpu.VMEM((_CH, _H), jnp.float32) for _ in range(_NB)]
          + [pltpu.VMEM_SHARED((_NPAD, _H), jnp.float32)]
          + [pltpu.SemaphoreType.DMA for _ in range(2 * _NB)]
      ),
  )
  def seg(p_hbm, src_hbm, dst_hbm, z_hbm, out_hbm, src_v, dst_v, *rest):
    rows = rest[:_NB]
    acc = rest[_NB]
    gsem = rest[_NB + 1:2 * _NB + 1]
    ssem = rest[2 * _NB + 1:]
    c = lax.axis_index("c")
    s = lax.axis_index("s")

    @pl.when(c == _WCORE)
    def _():
      pltpu.sync_copy(z_hbm.at[pl.ds(s * _RPT, _RPT)],
                      acc.at[pl.ds(s * _RPT, _RPT)])

    plsc.subcore_barrier()

    def run(base, nk):
      pltpu.sync_copy(src_hbm.at[pl.ds(base, nk)], src_v.at[pl.ds(0, nk)])
      pltpu.sync_copy(dst_hbm.at[pl.ds(base, nk)], dst_v.at[pl.ds(0, nk)])

      def body(t, carry):
        j0 = t * _NB
        gs = [pltpu.async_copy(p_hbm.at[src_v.at[j0 + b]], rows[b], gsem[b])
              for b in range(_NB)]
        ss = []
        for b in range(_NB):
          gs[b].wait()
          ss.append(pltpu.async_copy(rows[b], acc.at[dst_v.at[j0 + b]],
                                     ssem[b], add=True))
        for b in range(_NB):
          ss[b].wait()
        return carry

      lax.fori_loop(0, nk // _NB, body, 0)

    @pl.when(c == _WCORE)
    def _():
      run(s * _KW, _KW)

    plsc.subcore_barrier()

    @pl.when(c == _WCORE)
    def _():
      pltpu.sync_copy(acc.at[pl.ds(s * _RPT, _RPT)],
                      out_hbm.at[pl.ds(s * _RPT, _RPT)])

  return seg(p, src3, dst3, zeros)


def _stage0_body(x_ref, wr_ref, wo_ref, b_ref, p_ref, r_ref):
  x = x_ref[...]
  p_ref[...] = jnp.dot(x, wr_ref[...], preferred_element_type=jnp.float32)
  r_ref[...] = (jnp.dot(x, wo_ref[...], preferred_element_type=jnp.float32)
                + b_ref[...])


def _stage_mid_body(s_ref, r_ref, wr_ref, wo_ref, b_ref, p_ref, rn_ref):
  z = s_ref[:_N, :] + r_ref[...]
  h = jnp.where(z > 0, z, jnp.exp(z) - 1.0)
  p_ref[...] = jnp.dot(h, wr_ref[...], preferred_element_type=jnp.float32)
  rn_ref[...] = (jnp.dot(h, wo_ref[...], preferred_element_type=jnp.float32)
                 + b_ref[...])


def _stage_fin_body(s_ref, r_ref, bf_ref, wl_ref, bl_ref, out_ref):
  z = s_ref[:_N, :] + r_ref[...]
  h = jnp.where(z > 0, z, jnp.exp(z) - 1.0)
  gids = lax.broadcasted_iota(jnp.int32, (_G, _N), 0).astype(jnp.float32)
  onehot = (gids == bf_ref[...]).astype(jnp.float32)
  sums = jnp.dot(onehot, h, preferred_element_type=jnp.float32)
  counts = jnp.sum(onehot, axis=1, keepdims=True)
  pooled = sums / jnp.maximum(counts, 1.0)
  logits = (jnp.dot(pooled, wl_ref[...], preferred_element_type=jnp.float32)
            + bl_ref[...])
  m = jnp.max(logits, axis=1, keepdims=True)
  e = jnp.exp(logits - m)
  out_ref[...] = (logits - m) - jnp.log(jnp.sum(e, axis=1, keepdims=True))


def _sds(shape):
  return jax.ShapeDtypeStruct(shape, jnp.float32)


def kernel(x, edge_attr, W1_rel, b1_rel, W1_root, b1_root, W2_rel, b2_rel,
           W2_root, b2_root, W3_rel, b3_rel, W3_root, b3_root, W_lin, b_lin,
           edge_index, batch):
  del edge_attr
  f32 = jnp.float32
  pad = _EPAD - _E
  src3 = jnp.concatenate(
      [edge_index[0], jnp.zeros((pad,), jnp.int32)]).reshape(_TCHUNKS, _CH)
  dst3 = jnp.concatenate(
      [edge_index[1], jnp.full((pad,), _N, jnp.int32)]).reshape(_TCHUNKS, _CH)
  zeros = jnp.zeros((_NPAD, _H), f32)
  b1 = (b1_rel + b1_root).reshape(1, _H)
  b2 = (b2_rel + b2_root).reshape(1, _H)
  b3 = (b3_rel + b3_root).reshape(1, _H)
  batch_f = batch.astype(f32).reshape(1, _N)

  p1, r1 = pl.pallas_call(
      _stage0_body, out_shape=[_sds((_N, _H)), _sds((_N, _H))],
  )(x, W1_rel, W1_root, b1)
  s1 = _segment_sum_sc(p1, src3, dst3, zeros)

  p2, r2 = pl.pallas_call(
      _stage_mid_body, out_shape=[_sds((_N, _H)), _sds((_N, _H))],
  )(s1, r1, W2_rel, W2_root, b2)
  s2 = _segment_sum_sc(p2, src3, dst3, zeros)

  p3, r3 = pl.pallas_call(
      _stage_mid_body, out_shape=[_sds((_N, _H)), _sds((_N, _H))],
  )(s2, r2, W3_rel, W3_root, b3)
  s3 = _segment_sum_sc(p3, src3, dst3, zeros)

  out = pl.pallas_call(
      _stage_fin_body, out_shape=_sds((_G, _C)),
  )(s3, r3, batch_f, W_lin, b_lin.reshape(1, _C))
  return out


# speedup vs baseline: 2.2164x; 2.2164x over previous
import functools

import jax
import jax.numpy as jnp
from jax import lax
from jax.experimental import pallas as pl
from jax.experimental.pallas import tpu as pltpu
from jax.experimental.pallas import tpu_sc as plsc

_N = 10000
_E = 320000
_D = 128
_H = 32
_G = 64
_C = 2

_NTILES = 32
_CH = 128
_TCHUNKS = 2560
_EPAD = _TCHUNKS * _CH
_KW = 80
_NB = 8
_NPAD = 10112
_RPT = _NPAD // 16


def _segment_sum_sc(p, src3, dst3, zeros):
  mesh = plsc.VectorSubcoreMesh(core_axis_name="c", subcore_axis_name="s")

  @functools.partial(
      pl.kernel,
      mesh=mesh,
      compiler_params=pltpu.CompilerParams(use_tc_tiling_on_sc=False),
      out_type=jax.ShapeDtypeStruct((2, _NPAD, _H), jnp.float32),
      scratch_types=(
          [pltpu.VMEM((_KW, _CH), jnp.int32),
           pltpu.VMEM((_KW, _CH), jnp.int32)]
          + [pltpu.VMEM((_CH, _H), jnp.float32) for _ in range(_NB)]
          + [pltpu.VMEM_SHARED((_NPAD, _H), jnp.float32),
             pltpu.VMEM_SHARED((_N, _H), jnp.float32)]
          + [pltpu.SemaphoreType.DMA for _ in range(2 * _NB)]
      ),
  )
  def seg(p_hbm, src_hbm, dst_hbm, z_hbm, out_hbm, src_v, dst_v, *rest):
    rows = rest[:_NB]
    acc = rest[_NB]
    p_s = rest[_NB + 1]
    gsem = rest[_NB + 2:2 * _NB + 2]
    ssem = rest[2 * _NB + 2:]
    c = lax.axis_index("c")
    s = lax.axis_index("s")
    wid = c * 16 + s

    pltpu.sync_copy(z_hbm.at[pl.ds(s * _RPT, _RPT)],
                    acc.at[pl.ds(s * _RPT, _RPT)])

    @pl.when(s == 0)
    def _():
      pltpu.sync_copy(p_hbm, p_s)

    pltpu.sync_copy(src_hbm.at[pl.ds(wid * _KW, _KW)], src_v)
    pltpu.sync_copy(dst_hbm.at[pl.ds(wid * _KW, _KW)], dst_v)
    plsc.subcore_barrier()

    def body(t, carry):
      j0 = t * _NB
      gs = [pltpu.async_copy(p_s.at[src_v.at[j0 + b]], rows[b], gsem[b])
            for b in range(_NB)]
      ss = []
      for b in range(_NB):
        gs[b].wait()
        ss.append(pltpu.async_copy(rows[b], acc.at[dst_v.at[j0 + b]],
                                   ssem[b], add=True))
      for b in range(_NB):
        ss[b].wait()
      return carry

    lax.fori_loop(0, _KW // _NB, body, 0)
    plsc.subcore_barrier()
    pltpu.sync_copy(acc.at[pl.ds(s * _RPT, _RPT)],
                    out_hbm.at[c, pl.ds(s * _RPT, _RPT)])

  return seg(p, src3, dst3, zeros)


def _stage0_body(x_ref, wr_ref, wo_ref, b_ref, p_ref, r_ref):
  x = x_ref[...]
  p_ref[...] = jnp.dot(x, wr_ref[...], preferred_element_type=jnp.float32)
  r_ref[...] = (jnp.dot(x, wo_ref[...], preferred_element_type=jnp.float32)
                + b_ref[...])


def _stage_mid_body(s_ref, r_ref, wr_ref, wo_ref, b_ref, p_ref, rn_ref):
  z = s_ref[0, :_N, :] + s_ref[1, :_N, :] + r_ref[...]
  h = jnp.where(z > 0, z, jnp.exp(z) - 1.0)
  p_ref[...] = jnp.dot(h, wr_ref[...], preferred_element_type=jnp.float32)
  rn_ref[...] = (jnp.dot(h, wo_ref[...], preferred_element_type=jnp.float32)
                 + b_ref[...])


def _stage_fin_body(s_ref, r_ref, bf_ref, wl_ref, bl_ref, out_ref):
  z = s_ref[0, :_N, :] + s_ref[1, :_N, :] + r_ref[...]
  h = jnp.where(z > 0, z, jnp.exp(z) - 1.0)
  gids = lax.broadcasted_iota(jnp.int32, (_G, _N), 0).astype(jnp.float32)
  onehot = (gids == bf_ref[...]).astype(jnp.float32)
  sums = jnp.dot(onehot, h, preferred_element_type=jnp.float32)
  counts = jnp.sum(onehot, axis=1, keepdims=True)
  pooled = sums / jnp.maximum(counts, 1.0)
  logits = (jnp.dot(pooled, wl_ref[...], preferred_element_type=jnp.float32)
            + bl_ref[...])
  m = jnp.max(logits, axis=1, keepdims=True)
  e = jnp.exp(logits - m)
  out_ref[...] = (logits - m) - jnp.log(jnp.sum(e, axis=1, keepdims=True))


def _sds(shape):
  return jax.ShapeDtypeStruct(shape, jnp.float32)


def kernel(x, edge_attr, W1_rel, b1_rel, W1_root, b1_root, W2_rel, b2_rel,
           W2_root, b2_root, W3_rel, b3_rel, W3_root, b3_root, W_lin, b_lin,
           edge_index, batch):
  del edge_attr
  f32 = jnp.float32
  pad = _EPAD - _E
  src3 = jnp.concatenate(
      [edge_index[0], jnp.zeros((pad,), jnp.int32)]).reshape(_TCHUNKS, _CH)
  dst3 = jnp.concatenate(
      [edge_index[1], jnp.full((pad,), _N, jnp.int32)]).reshape(_TCHUNKS, _CH)
  zeros = jnp.zeros((_NPAD, _H), f32)
  b1 = (b1_rel + b1_root).reshape(1, _H)
  b2 = (b2_rel + b2_root).reshape(1, _H)
  b3 = (b3_rel + b3_root).reshape(1, _H)
  batch_f = batch.astype(f32).reshape(1, _N)

  p1, r1 = pl.pallas_call(
      _stage0_body, out_shape=[_sds((_N, _H)), _sds((_N, _H))],
  )(x, W1_rel, W1_root, b1)
  s1 = _segment_sum_sc(p1, src3, dst3, zeros)

  p2, r2 = pl.pallas_call(
      _stage_mid_body, out_shape=[_sds((_N, _H)), _sds((_N, _H))],
  )(s1, r1, W2_rel, W2_root, b2)
  s2 = _segment_sum_sc(p2, src3, dst3, zeros)

  p3, r3 = pl.pallas_call(
      _stage_mid_body, out_shape=[_sds((_N, _H)), _sds((_N, _H))],
  )(s2, r2, W3_rel, W3_root, b3)
  s3 = _segment_sum_sc(p3, src3, dst3, zeros)

  out = pl.pallas_call(
      _stage_fin_body, out_shape=_sds((_G, _C)),
  )(s3, r3, batch_f, W_lin, b_lin.reshape(1, _C))
  return out


# speedup vs baseline: 2.3082x; 1.0414x over previous
import functools

import jax
import jax.numpy as jnp
from jax import lax
from jax.experimental import pallas as pl
from jax.experimental.pallas import tpu as pltpu
from jax.experimental.pallas import tpu_sc as plsc

_N = 10000
_E = 320000
_D = 128
_H = 32
_G = 64
_C = 2

_NTILES = 32
_CH = 128
_TCHUNKS = 2560
_EPAD = _TCHUNKS * _CH
_KW = 80
_NB = 8
_NPAD = 10112
_RPT = _NPAD // 16


def _segment_sum_sc(p, ei3, zeros):
  mesh = plsc.VectorSubcoreMesh(core_axis_name="c", subcore_axis_name="s")

  @functools.partial(
      pl.kernel,
      mesh=mesh,
      compiler_params=pltpu.CompilerParams(use_tc_tiling_on_sc=False),
      out_type=jax.ShapeDtypeStruct((2, _NPAD, _H), jnp.float32),
      scratch_types=(
          [pltpu.VMEM((_KW, _CH), jnp.int32),
           pltpu.VMEM((_KW, _CH), jnp.int32)]
          + [pltpu.VMEM((_CH, _H), jnp.float32) for _ in range(_NB)]
          + [pltpu.VMEM_SHARED((_NPAD, _H), jnp.float32),
             pltpu.VMEM_SHARED((_N + 16, _H), jnp.float32)]
          + [pltpu.SemaphoreType.DMA for _ in range(2 * _NB)]
      ),
  )
  def seg(p_hbm, ei_hbm, z_hbm, out_hbm, src_v, dst_v, *rest):
    rows = rest[:_NB]
    acc = rest[_NB]
    p_s = rest[_NB + 1]
    gsem = rest[_NB + 2:2 * _NB + 2]
    ssem = rest[2 * _NB + 2:]
    c = lax.axis_index("c")
    s = lax.axis_index("s")
    wid = c * 16 + s

    pltpu.sync_copy(z_hbm.at[pl.ds(s * _RPT, _RPT)],
                    acc.at[pl.ds(s * _RPT, _RPT)])

    @pl.when(s == 0)
    def _():
      pltpu.sync_copy(p_hbm, p_s.at[pl.ds(0, _N)])

    pltpu.sync_copy(ei_hbm.at[0, pl.ds(wid * _KW, _KW)], src_v)
    pltpu.sync_copy(ei_hbm.at[1, pl.ds(wid * _KW, _KW)], dst_v)
    plsc.subcore_barrier()

    def body(t, carry):
      j0 = t * _NB
      gs = [pltpu.async_copy(p_s.at[src_v.at[j0 + b]], rows[b], gsem[b])
            for b in range(_NB)]
      ss = []
      for b in range(_NB):
        gs[b].wait()
        ss.append(pltpu.async_copy(rows[b], acc.at[dst_v.at[j0 + b]],
                                   ssem[b], add=True))
      for b in range(_NB):
        ss[b].wait()
      return carry

    lax.fori_loop(0, _KW // _NB, body, 0)
    plsc.subcore_barrier()
    pltpu.sync_copy(acc.at[pl.ds(s * _RPT, _RPT)],
                    out_hbm.at[c, pl.ds(s * _RPT, _RPT)])

  return seg(p, ei3, zeros)


def _stage0_body(x_ref, wr_ref, wo_ref, b_ref, p_ref, r_ref):
  x = x_ref[...]
  p_ref[...] = jnp.dot(x, wr_ref[...], preferred_element_type=jnp.float32)
  r_ref[...] = (jnp.dot(x, wo_ref[...], preferred_element_type=jnp.float32)
                + b_ref[...])


def _stage_mid_body(s_ref, r_ref, wr_ref, wo_ref, b_ref, p_ref, rn_ref):
  z = s_ref[0, :_N, :] + s_ref[1, :_N, :] + r_ref[...]
  h = jnp.where(z > 0, z, jnp.exp(z) - 1.0)
  p_ref[...] = jnp.dot(h, wr_ref[...], preferred_element_type=jnp.float32)
  rn_ref[...] = (jnp.dot(h, wo_ref[...], preferred_element_type=jnp.float32)
                 + b_ref[...])


def _stage_fin_body(s_ref, r_ref, bf_ref, wl_ref, bl_ref, out_ref):
  z = s_ref[0, :_N, :] + s_ref[1, :_N, :] + r_ref[...]
  h = jnp.where(z > 0, z, jnp.exp(z) - 1.0)
  gids = lax.broadcasted_iota(jnp.int32, (_G, _N), 0).astype(jnp.float32)
  onehot = (gids == bf_ref[...]).astype(jnp.float32)
  sums = jnp.dot(onehot, h, preferred_element_type=jnp.float32)
  counts = jnp.sum(onehot, axis=1, keepdims=True)
  pooled = sums / jnp.maximum(counts, 1.0)
  logits = (jnp.dot(pooled, wl_ref[...], preferred_element_type=jnp.float32)
            + bl_ref[...])
  m = jnp.max(logits, axis=1, keepdims=True)
  e = jnp.exp(logits - m)
  out_ref[...] = (logits - m) - jnp.log(jnp.sum(e, axis=1, keepdims=True))


def _sds(shape):
  return jax.ShapeDtypeStruct(shape, jnp.float32)


def kernel(x, edge_attr, W1_rel, b1_rel, W1_root, b1_root, W2_rel, b2_rel,
           W2_root, b2_root, W3_rel, b3_rel, W3_root, b3_root, W_lin, b_lin,
           edge_index, batch):
  del edge_attr
  f32 = jnp.float32
  ei3 = jnp.pad(edge_index, ((0, 0), (0, _EPAD - _E)),
                constant_values=_N).reshape(2, _TCHUNKS, _CH)
  zeros = jnp.zeros((_NPAD, _H), f32)
  b1 = (b1_rel + b1_root).reshape(1, _H)
  b2 = (b2_rel + b2_root).reshape(1, _H)
  b3 = (b3_rel + b3_root).reshape(1, _H)
  batch_f = batch.astype(f32).reshape(1, _N)

  p1, r1 = pl.pallas_call(
      _stage0_body, out_shape=[_sds((_N, _H)), _sds((_N, _H))],
  )(x, W1_rel, W1_root, b1)
  s1 = _segment_sum_sc(p1, ei3, zeros)

  p2, r2 = pl.pallas_call(
      _stage_mid_body, out_shape=[_sds((_N, _H)), _sds((_N, _H))],
  )(s1, r1, W2_rel, W2_root, b2)
  s2 = _segment_sum_sc(p2, ei3, zeros)

  p3, r3 = pl.pallas_call(
      _stage_mid_body, out_shape=[_sds((_N, _H)), _sds((_N, _H))],
  )(s2, r2, W3_rel, W3_root, b3)
  s3 = _segment_sum_sc(p3, ei3, zeros)

  out = pl.pallas_call(
      _stage_fin_body, out_shape=_sds((_G, _C)),
  )(s3, r3, batch_f, W_lin, b_lin.reshape(1, _C))
  return out


# speedup vs baseline: 2.3886x; 1.0348x over previous
import functools

import jax
import jax.numpy as jnp
from jax import lax
from jax.experimental import pallas as pl
from jax.experimental.pallas import tpu as pltpu
from jax.experimental.pallas import tpu_sc as plsc

_N = 10000
_E = 320000
_D = 128
_H = 32
_G = 64
_C = 2

_NTILES = 32
_CH = 128
_TCHUNKS = 2560
_EPAD = _TCHUNKS * _CH
_K0 = 88
_K1 = 72
_NB = 8
_NPAD = 10112
_RPT = _NPAD // 16


def _segment_sum_sc(p, ei3, zeros):
  mesh = plsc.VectorSubcoreMesh(core_axis_name="c", subcore_axis_name="s")

  @functools.partial(
      pl.kernel,
      mesh=mesh,
      compiler_params=pltpu.CompilerParams(use_tc_tiling_on_sc=False),
      out_type=jax.ShapeDtypeStruct((2, _NPAD, _H), jnp.float32),
      scratch_types=(
          [pltpu.VMEM((_K0, _CH), jnp.int32),
           pltpu.VMEM((_K0, _CH), jnp.int32)]
          + [pltpu.VMEM((_CH, _H), jnp.float32) for _ in range(_NB)]
          + [pltpu.VMEM_SHARED((_NPAD, _H), jnp.float32),
             pltpu.VMEM_SHARED((_N + 16, _H), jnp.float32)]
          + [pltpu.SemaphoreType.DMA for _ in range(2 * _NB)]
      ),
  )
  def seg(p_hbm, ei_hbm, z_hbm, out_hbm, src_v, dst_v, *rest):
    rows = rest[:_NB]
    acc = rest[_NB]
    p_s = rest[_NB + 1]
    gsem = rest[_NB + 2:2 * _NB + 2]
    ssem = rest[2 * _NB + 2:]
    c = lax.axis_index("c")
    s = lax.axis_index("s")
    wid = c * 16 + s

    pltpu.sync_copy(z_hbm.at[pl.ds(s * _RPT, _RPT)],
                    acc.at[pl.ds(s * _RPT, _RPT)])

    @pl.when(s == 0)
    def _():
      pltpu.sync_copy(p_hbm, p_s.at[pl.ds(0, _N)])

    plsc.subcore_barrier()

    def run(base, nk):
      pltpu.sync_copy(ei_hbm.at[0, pl.ds(base, nk)], src_v.at[pl.ds(0, nk)])
      pltpu.sync_copy(ei_hbm.at[1, pl.ds(base, nk)], dst_v.at[pl.ds(0, nk)])

      def body(t, carry):
        j0 = t * _NB
        gs = [pltpu.async_copy(p_s.at[src_v.at[j0 + b]], rows[b], gsem[b])
              for b in range(_NB)]
        ss = []
        for b in range(_NB):
          gs[b].wait()
          ss.append(pltpu.async_copy(rows[b], acc.at[dst_v.at[j0 + b]],
                                     ssem[b], add=True))
        for b in range(_NB):
          ss[b].wait()
        return carry

      lax.fori_loop(0, nk // _NB, body, 0)

    @pl.when(c == 0)
    def _():
      run(s * _K0, _K0)

    @pl.when(c == 1)
    def _():
      run(16 * _K0 + s * _K1, _K1)

    plsc.subcore_barrier()
    pltpu.sync_copy(acc.at[pl.ds(s * _RPT, _RPT)],
                    out_hbm.at[c, pl.ds(s * _RPT, _RPT)])

  return seg(p, ei3, zeros)


def _stage0_body(x_ref, wr_ref, wo_ref, b_ref, p_ref, r_ref):
  x = x_ref[...]
  p_ref[...] = jnp.dot(x, wr_ref[...], preferred_element_type=jnp.float32)
  r_ref[...] = (jnp.dot(x, wo_ref[...], preferred_element_type=jnp.float32)
                + b_ref[...])


def _stage_mid_body(s_ref, r_ref, wr_ref, wo_ref, b_ref, p_ref, rn_ref):
  z = s_ref[0, :_N, :] + s_ref[1, :_N, :] + r_ref[...]
  h = jnp.where(z > 0, z, jnp.exp(z) - 1.0)
  p_ref[...] = jnp.dot(h, wr_ref[...], preferred_element_type=jnp.float32)
  rn_ref[...] = (jnp.dot(h, wo_ref[...], preferred_element_type=jnp.float32)
                 + b_ref[...])


def _stage_fin_body(s_ref, r_ref, bf_ref, wl_ref, bl_ref, out_ref):
  z = s_ref[0, :_N, :] + s_ref[1, :_N, :] + r_ref[...]
  h = jnp.where(z > 0, z, jnp.exp(z) - 1.0)
  gids = lax.broadcasted_iota(jnp.int32, (_G, _N), 0).astype(jnp.float32)
  onehot = (gids == bf_ref[...]).astype(jnp.float32)
  sums = jnp.dot(onehot, h, preferred_element_type=jnp.float32)
  counts = jnp.sum(onehot, axis=1, keepdims=True)
  pooled = sums / jnp.maximum(counts, 1.0)
  logits = (jnp.dot(pooled, wl_ref[...], preferred_element_type=jnp.float32)
            + bl_ref[...])
  m = jnp.max(logits, axis=1, keepdims=True)
  e = jnp.exp(logits - m)
  out_ref[...] = (logits - m) - jnp.log(jnp.sum(e, axis=1, keepdims=True))


def _sds(shape):
  return jax.ShapeDtypeStruct(shape, jnp.float32)


def kernel(x, edge_attr, W1_rel, b1_rel, W1_root, b1_root, W2_rel, b2_rel,
           W2_root, b2_root, W3_rel, b3_rel, W3_root, b3_root, W_lin, b_lin,
           edge_index, batch):
  del edge_attr
  f32 = jnp.float32
  ei3 = jnp.pad(edge_index, ((0, 0), (0, _EPAD - _E)),
                constant_values=_N).reshape(2, _TCHUNKS, _CH)
  zeros = jnp.zeros((_NPAD, _H), f32)
  b1 = (b1_rel + b1_root).reshape(1, _H)
  b2 = (b2_rel + b2_root).reshape(1, _H)
  b3 = (b3_rel + b3_root).reshape(1, _H)
  batch_f = batch.astype(f32).reshape(1, _N)

  p1, r1 = pl.pallas_call(
      _stage0_body, out_shape=[_sds((_N, _H)), _sds((_N, _H))],
  )(x, W1_rel, W1_root, b1)
  s1 = _segment_sum_sc(p1, ei3, zeros)

  p2, r2 = pl.pallas_call(
      _stage_mid_body, out_shape=[_sds((_N, _H)), _sds((_N, _H))],
  )(s1, r1, W2_rel, W2_root, b2)
  s2 = _segment_sum_sc(p2, ei3, zeros)

  p3, r3 = pl.pallas_call(
      _stage_mid_body, out_shape=[_sds((_N, _H)), _sds((_N, _H))],
  )(s2, r2, W3_rel, W3_root, b3)
  s3 = _segment_sum_sc(p3, ei3, zeros)

  out = pl.pallas_call(
      _stage_fin_body, out_shape=_sds((_G, _C)),
  )(s3, r3, batch_f, W_lin, b_lin.reshape(1, _C))
  return out


# speedup vs baseline: 2.4354x; 1.0196x over previous
import functools

import jax
import jax.numpy as jnp
from jax import lax
from jax.experimental import pallas as pl
from jax.experimental.pallas import tpu as pltpu
from jax.experimental.pallas import tpu_sc as plsc

_N = 10000
_E = 320000
_D = 128
_H = 32
_G = 64
_C = 2

_NTILES = 32
_CH = 128
_TCHUNKS = 2560
_EPAD = _TCHUNKS * _CH
_K0 = 96
_K1 = 64
_NB = 8
_NPAD = 10112
_RPT = _NPAD // 16


def _segment_sum_sc(p, ei3, zeros):
  mesh = plsc.VectorSubcoreMesh(core_axis_name="c", subcore_axis_name="s")

  @functools.partial(
      pl.kernel,
      mesh=mesh,
      compiler_params=pltpu.CompilerParams(use_tc_tiling_on_sc=False),
      out_type=jax.ShapeDtypeStruct((2, _NPAD, _H), jnp.float32),
      scratch_types=(
          [pltpu.VMEM((_K0, _CH), jnp.int32),
           pltpu.VMEM((_K0, _CH), jnp.int32)]
          + [pltpu.VMEM((_CH, _H), jnp.float32) for _ in range(_NB)]
          + [pltpu.VMEM_SHARED((_NPAD, _H), jnp.float32),
             pltpu.VMEM_SHARED((_N + 16, _H), jnp.float32)]
          + [pltpu.SemaphoreType.DMA for _ in range(2 * _NB)]
      ),
  )
  def seg(p_hbm, ei_hbm, z_hbm, out_hbm, src_v, dst_v, *rest):
    rows = rest[:_NB]
    acc = rest[_NB]
    p_s = rest[_NB + 1]
    gsem = rest[_NB + 2:2 * _NB + 2]
    ssem = rest[2 * _NB + 2:]
    c = lax.axis_index("c")
    s = lax.axis_index("s")
    wid = c * 16 + s

    pltpu.sync_copy(z_hbm.at[pl.ds(s * _RPT, _RPT)],
                    acc.at[pl.ds(s * _RPT, _RPT)])

    @pl.when(s == 0)
    def _():
      pltpu.sync_copy(p_hbm, p_s.at[pl.ds(0, _N)])

    plsc.subcore_barrier()

    def run(base, nk):
      pltpu.sync_copy(ei_hbm.at[0, pl.ds(base, nk)], src_v.at[pl.ds(0, nk)])
      pltpu.sync_copy(ei_hbm.at[1, pl.ds(base, nk)], dst_v.at[pl.ds(0, nk)])

      def body(t, carry):
        j0 = t * _NB
        gs = [pltpu.async_copy(p_s.at[src_v.at[j0 + b]], rows[b], gsem[b])
              for b in range(_NB)]
        ss = []
        for b in range(_NB):
          gs[b].wait()
          ss.append(pltpu.async_copy(rows[b], acc.at[dst_v.at[j0 + b]],
                                     ssem[b], add=True))
        for b in range(_NB):
          ss[b].wait()
        return carry

      lax.fori_loop(0, nk // _NB, body, 0)

    @pl.when(c == 0)
    def _():
      run(s * _K0, _K0)

    @pl.when(c == 1)
    def _():
      run(16 * _K0 + s * _K1, _K1)

    plsc.subcore_barrier()
    pltpu.sync_copy(acc.at[pl.ds(s * _RPT, _RPT)],
                    out_hbm.at[c, pl.ds(s * _RPT, _RPT)])

  return seg(p, ei3, zeros)


def _stage0_body(x_ref, wr_ref, wo_ref, b_ref, p_ref, r_ref):
  x = x_ref[...]
  p_ref[...] = jnp.dot(x, wr_ref[...], preferred_element_type=jnp.float32)
  r_ref[...] = (jnp.dot(x, wo_ref[...], preferred_element_type=jnp.float32)
                + b_ref[...])


def _stage_mid_body(s_ref, r_ref, wr_ref, wo_ref, b_ref, p_ref, rn_ref):
  z = s_ref[0, :_N, :] + s_ref[1, :_N, :] + r_ref[...]
  h = jnp.where(z > 0, z, jnp.exp(z) - 1.0)
  p_ref[...] = jnp.dot(h, wr_ref[...], preferred_element_type=jnp.float32)
  rn_ref[...] = (jnp.dot(h, wo_ref[...], preferred_element_type=jnp.float32)
                 + b_ref[...])


def _stage_fin_body(s_ref, r_ref, bf_ref, wl_ref, bl_ref, out_ref):
  z = s_ref[0, :_N, :] + s_ref[1, :_N, :] + r_ref[...]
  h = jnp.where(z > 0, z, jnp.exp(z) - 1.0)
  gids = lax.broadcasted_iota(jnp.int32, (_G, _N), 0).astype(jnp.float32)
  onehot = (gids == bf_ref[...]).astype(jnp.float32)
  sums = jnp.dot(onehot, h, preferred_element_type=jnp.float32)
  counts = jnp.sum(onehot, axis=1, keepdims=True)
  pooled = sums / jnp.maximum(counts, 1.0)
  logits = (jnp.dot(pooled, wl_ref[...], preferred_element_type=jnp.float32)
            + bl_ref[...])
  m = jnp.max(logits, axis=1, keepdims=True)
  e = jnp.exp(logits - m)
  out_ref[...] = (logits - m) - jnp.log(jnp.sum(e, axis=1, keepdims=True))


def _sds(shape):
  return jax.ShapeDtypeStruct(shape, jnp.float32)


def kernel(x, edge_attr, W1_rel, b1_rel, W1_root, b1_root, W2_rel, b2_rel,
           W2_root, b2_root, W3_rel, b3_rel, W3_root, b3_root, W_lin, b_lin,
           edge_index, batch):
  del edge_attr
  f32 = jnp.float32
  ei3 = jnp.pad(edge_index, ((0, 0), (0, _EPAD - _E)),
                constant_values=_N).reshape(2, _TCHUNKS, _CH)
  zeros = jnp.zeros((_NPAD, _H), f32)
  b1 = (b1_rel + b1_root).reshape(1, _H)
  b2 = (b2_rel + b2_root).reshape(1, _H)
  b3 = (b3_rel + b3_root).reshape(1, _H)
  batch_f = batch.astype(f32).reshape(1, _N)

  p1, r1 = pl.pallas_call(
      _stage0_body, out_shape=[_sds((_N, _H)), _sds((_N, _H))],
  )(x, W1_rel, W1_root, b1)
  s1 = _segment_sum_sc(p1, ei3, zeros)

  p2, r2 = pl.pallas_call(
      _stage_mid_body, out_shape=[_sds((_N, _H)), _sds((_N, _H))],
  )(s1, r1, W2_rel, W2_root, b2)
  s2 = _segment_sum_sc(p2, ei3, zeros)

  p3, r3 = pl.pallas_call(
      _stage_mid_body, out_shape=[_sds((_N, _H)), _sds((_N, _H))],
  )(s2, r2, W3_rel, W3_root, b3)
  s3 = _segment_sum_sc(p3, ei3, zeros)

  out = pl.pallas_call(
      _stage_fin_body, out_shape=_sds((_G, _C)),
  )(s3, r3, batch_f, W_lin, b_lin.reshape(1, _C))
  return out
